# baseline (speedup 1.0000x reference)
import functools

import jax
import jax.numpy as jnp
from jax import lax
from jax.experimental import pallas as pl
from jax.experimental.pallas import tpu as pltpu
from jax.experimental.pallas import tpu_sc as plsc

_B = 16384
_D = 64
_NW = 32
_BPW = _B // _NW
_CHUNK = 128
_NCHUNK = _BPW // _CHUNK
_NT = 100000


_CROWS = 400
_NCC = _NT // _CROWS
_CFULL = 8


def _conv_one(table_hbm, out_hbm, nc, cb, stage_v, cout_v, sem):
    def chunk(c, carry):
        g = cb + c
        pltpu.sync_copy(
            table_hbm.at[pl.ds(pl.multiple_of(g * _CROWS, 8), _CROWS)],
            stage_v)

        def pairs(pp, carry2):
            for q in range(4):
                p = pp * 4 + q
                for k in range(_D // 16):
                    cout_v[p, pl.ds(k * 16, 16)] = \
                        stage_v[2 * p, pl.ds(k * 16, 16)]
                    cout_v[p, pl.ds(_D + k * 16, 16)] = \
                        stage_v[2 * p + 1, pl.ds(k * 16, 16)]
            return carry2

        lax.fori_loop(0, _CROWS // 8, pairs, 0)
        pltpu.sync_copy(
            cout_v,
            out_hbm.at[pl.ds(pl.multiple_of(g * (_CROWS // 2), 8),
                             _CROWS // 2)])
        return carry

    lax.fori_loop(0, nc, chunk, 0)


def _conv_body(user_hbm, item_hbm, user2_hbm, item2_hbm,
               stage_v, cout_v, sem):
    wid = lax.axis_index("s") * 2 + lax.axis_index("c")
    extra = _NCC - _NW * (_CFULL - 1)
    nc = jnp.where(wid < extra, _CFULL, _CFULL - 1)
    cb = wid * (_CFULL - 1) + jnp.minimum(wid, extra)
    _conv_one(user_hbm, user2_hbm, nc, cb, stage_v, cout_v, sem)
    _conv_one(item_hbm, item2_hbm, nc, cb, stage_v, cout_v, sem)


def _sc_body(inp_hbm, user_hbm, item_hbm, pout_hbm,
             inp_v, uidx_v, iidx_v, uoff_v, ioff_v, urows_v, irows_v,
             pout_v, sem):
    wid = lax.axis_index("s") * 2 + lax.axis_index("c")
    base = wid * _BPW
    lane = lax.iota(jnp.int32, 16)
    lo = lane < 8
    idx_a = jnp.where(lo, lane * 2, 0)
    idx_b = jnp.where(lo, 0, (lane - 8) * 2)

    pltpu.sync_copy(
        inp_hbm.at[pl.ds(pl.multiple_of(base * 2, 8), _BPW * 2)], inp_v)

    def deint(j, carry):
        a = inp_v[pl.ds(j * 32, 16)]
        b = inp_v[pl.ds(j * 32 + 16, 16)]
        ua = a.at[idx_a].get(mode="promise_in_bounds")
        ub = b.at[idx_b].get(mode="promise_in_bounds")
        ia = a.at[idx_a + 1].get(mode="promise_in_bounds")
        ib = b.at[idx_b + 1].get(mode="promise_in_bounds")
        u = jnp.where(lo, ua, ub)
        i = jnp.where(lo, ia, ib)
        uidx_v[pl.ds(j * 16, 16)] = u >> 1
        iidx_v[pl.ds(j * 16, 16)] = i >> 1
        uoff_v[pl.ds(j * 16, 16)] = (u & 1) * _D
        ioff_v[pl.ds(j * 16, 16)] = (i & 1) * _D
        return carry

    lax.fori_loop(0, _BPW // 16, deint, 0)

    def chunk(c, carry):
        cu = pltpu.make_async_copy(
            user_hbm.at[uidx_v.at[pl.ds(pl.multiple_of(c * _CHUNK, 8),
                                        _CHUNK)]], urows_v, sem)
        ci = pltpu.make_async_copy(
            item_hbm.at[iidx_v.at[pl.ds(pl.multiple_of(c * _CHUNK, 8),
                                        _CHUNK)]], irows_v, sem)
        cu.start()
        ci.start()
        cu.wait()
        ci.wait()

        def body(g, carry2):
            uoffs = uoff_v[pl.ds(c * _CHUNK + g * 16, 16)]
            ioffs = ioff_v[pl.ds(c * _CHUNK + g * 16, 16)]
            for rr in range(16):
                r = g * 16 + rr
                uoff = uoffs[rr]
                ioff = ioffs[rr]
                s = (urows_v[r, pl.ds(uoff, 16)] *
                     irows_v[r, pl.ds(ioff, 16)])
                for k in range(1, _D // 16):
                    s = s + (urows_v[r, pl.ds(uoff + k * 16, 16)] *
                             irows_v[r, pl.ds(ioff + k * 16, 16)])
                row = c * 16 + g * 2 + rr // 8
                pout_v[row, pl.ds((rr % 8) * 16, 16)] = s
            return carry2

        lax.fori_loop(0, _CHUNK // 16, body, 0)
        return carry

    lax.fori_loop(0, _NCHUNK, chunk, 0)
    pltpu.sync_copy(
        pout_v,
        pout_hbm.at[pl.ds(pl.multiple_of(wid * (_BPW // 8), 8), _BPW // 8)])


def _tc_body(p_ref, o_ref):
    r = lax.broadcasted_iota(jnp.int32, (128, 8), 0)
    c = lax.broadcasted_iota(jnp.int32, (128, 8), 1)
    sel = (r // 16 == c).astype(jnp.float32)
    o_ref[...] = jnp.dot(p_ref[...], sel, preferred_element_type=jnp.float32)


def kernel(inputs, user_embedding, item_embedding):
    mesh = plsc.VectorSubcoreMesh(core_axis_name="c", subcore_axis_name="s")
    conv_run = functools.partial(
        pl.kernel,
        out_type=(
            jax.ShapeDtypeStruct((_NT // 2, 2 * _D), jnp.float32),
            jax.ShapeDtypeStruct((_NT // 2, 2 * _D), jnp.float32),
        ),
        mesh=mesh,
        scratch_types=[
            pltpu.VMEM((_CROWS, _D), jnp.float32),
            pltpu.VMEM((_CROWS // 2, 2 * _D), jnp.float32),
            pltpu.SemaphoreType.DMA,
        ],
    )(_conv_body)
    sc_run = functools.partial(
        pl.kernel,
        out_type=jax.ShapeDtypeStruct((_B // 8, 128), jnp.float32),
        mesh=mesh,
        scratch_types=[
            pltpu.VMEM((_BPW * 2,), jnp.int32),
            pltpu.VMEM((_BPW,), jnp.int32),
            pltpu.VMEM((_BPW,), jnp.int32),
            pltpu.VMEM((_BPW,), jnp.int32),
            pltpu.VMEM((_BPW,), jnp.int32),
            pltpu.VMEM((_CHUNK, 2 * _D), jnp.float32),
            pltpu.VMEM((_CHUNK, 2 * _D), jnp.float32),
            pltpu.VMEM((_BPW // 8, 128), jnp.float32),
            pltpu.SemaphoreType.DMA,
        ],
    )(_sc_body)
    user2, item2 = conv_run(user_embedding, item_embedding)
    partials = sc_run(inputs.reshape(_B * 2), user2, item2)
    out = pl.pallas_call(
        _tc_body,
        out_shape=jax.ShapeDtypeStruct((_B // 8, 8), jnp.float32),
    )(partials)
    return out.reshape(_B)


# speedup vs baseline: 1.2298x; 1.2298x over previous
import functools

import jax
import jax.numpy as jnp
from jax import lax
from jax.experimental import pallas as pl
from jax.experimental.pallas import tpu as pltpu
from jax.experimental.pallas import tpu_sc as plsc

_B = 16384
_D = 64
_NW = 32
_BPW = _B // _NW
_CHUNK = 128
_NCHUNK = _BPW // _CHUNK
_NT = 100000


_CROWS = 80
_NCC = _NT // _CROWS
_CFULL = 40


def _compact(stage_v, cout_v):
    def pairs(pp, carry2):
        for q in range(4):
            p = pp * 4 + q
            for k in range(_D // 16):
                cout_v[p, pl.ds(k * 16, 16)] = \
                    stage_v[2 * p, pl.ds(k * 16, 16)]
                cout_v[p, pl.ds(_D + k * 16, 16)] = \
                    stage_v[2 * p + 1, pl.ds(k * 16, 16)]
        return carry2

    lax.fori_loop(0, _CROWS // 8, pairs, 0)


def _conv_one(table_hbm, out_hbm, nc, cb, stage, cout, sem_in, sem_out):
    def in_copy(c, b):
        g = cb + c
        return pltpu.make_async_copy(
            table_hbm.at[pl.ds(pl.multiple_of(g * _CROWS, 8), _CROWS)],
            stage[b], sem_in[b])

    def out_copy(c, b):
        g = cb + c
        return pltpu.make_async_copy(
            cout[b],
            out_hbm.at[pl.ds(pl.multiple_of(g * (_CROWS // 2), 8),
                             _CROWS // 2)],
            sem_out[b])

    in_copy(0, 0).start()

    def pair(j, carry):
        for b in range(2):
            c = 2 * j + b

            @pl.when(c < nc)
            def _():
                in_copy(c, b).wait()

                @pl.when(c + 1 < nc)
                def _():
                    in_copy(c + 1, 1 - b).start()

                @pl.when(c >= 2)
                def _():
                    out_copy(c - 2, b).wait()

                _compact(stage[b], cout[b])
                out_copy(c, b).start()
        return carry

    lax.fori_loop(0, _CFULL // 2, pair, 0)
    out_copy(0, 0).wait()
    out_copy(1, 1).wait()


def _conv_body(user_hbm, item_hbm, user2_hbm, item2_hbm,
               stage0, stage1, cout0, cout1,
               sem_in0, sem_in1, sem_out0, sem_out1):
    wid = lax.axis_index("s") * 2 + lax.axis_index("c")
    extra = _NCC - _NW * (_CFULL - 1)
    nc = jnp.where(wid < extra, _CFULL, _CFULL - 1)
    cb = wid * (_CFULL - 1) + jnp.minimum(wid, extra)
    stage = (stage0, stage1)
    cout = (cout0, cout1)
    sem_in = (sem_in0, sem_in1)
    sem_out = (sem_out0, sem_out1)
    _conv_one(user_hbm, user2_hbm, nc, cb, stage, cout, sem_in, sem_out)
    _conv_one(item_hbm, item2_hbm, nc, cb, stage, cout, sem_in, sem_out)


def _sc_body(inp_hbm, user_hbm, item_hbm, pout_hbm,
             inp_v, uidx_v, iidx_v, uoff_v, ioff_v, urows_v, irows_v,
             pout_v, sem):
    wid = lax.axis_index("s") * 2 + lax.axis_index("c")
    base = wid * _BPW
    lane = lax.iota(jnp.int32, 16)
    lo = lane < 8
    idx_a = jnp.where(lo, lane * 2, 0)
    idx_b = jnp.where(lo, 0, (lane - 8) * 2)

    pltpu.sync_copy(
        inp_hbm.at[pl.ds(pl.multiple_of(base * 2, 8), _BPW * 2)], inp_v)

    def deint(j, carry):
        a = inp_v[pl.ds(j * 32, 16)]
        b = inp_v[pl.ds(j * 32 + 16, 16)]
        ua = a.at[idx_a].get(mode="promise_in_bounds")
        ub = b.at[idx_b].get(mode="promise_in_bounds")
        ia = a.at[idx_a + 1].get(mode="promise_in_bounds")
        ib = b.at[idx_b + 1].get(mode="promise_in_bounds")
        u = jnp.where(lo, ua, ub)
        i = jnp.where(lo, ia, ib)
        uidx_v[pl.ds(j * 16, 16)] = u >> 1
        iidx_v[pl.ds(j * 16, 16)] = i >> 1
        uoff_v[pl.ds(j * 16, 16)] = (u & 1) * _D
        ioff_v[pl.ds(j * 16, 16)] = (i & 1) * _D
        return carry

    lax.fori_loop(0, _BPW // 16, deint, 0)

    def chunk(c, carry):
        cu = pltpu.make_async_copy(
            user_hbm.at[uidx_v.at[pl.ds(pl.multiple_of(c * _CHUNK, 8),
                                        _CHUNK)]], urows_v, sem)
        ci = pltpu.make_async_copy(
            item_hbm.at[iidx_v.at[pl.ds(pl.multiple_of(c * _CHUNK, 8),
                                        _CHUNK)]], irows_v, sem)
        cu.start()
        ci.start()
        cu.wait()
        ci.wait()

        def body(g, carry2):
            uoffs = uoff_v[pl.ds(c * _CHUNK + g * 16, 16)]
            ioffs = ioff_v[pl.ds(c * _CHUNK + g * 16, 16)]
            for rr in range(16):
                r = g * 16 + rr
                uoff = uoffs[rr]
                ioff = ioffs[rr]
                s = (urows_v[r, pl.ds(uoff, 16)] *
                     irows_v[r, pl.ds(ioff, 16)])
                for k in range(1, _D // 16):
                    s = s + (urows_v[r, pl.ds(uoff + k * 16, 16)] *
                             irows_v[r, pl.ds(ioff + k * 16, 16)])
                row = c * 16 + g * 2 + rr // 8
                pout_v[row, pl.ds((rr % 8) * 16, 16)] = s
            return carry2

        lax.fori_loop(0, _CHUNK // 16, body, 0)
        return carry

    lax.fori_loop(0, _NCHUNK, chunk, 0)
    pltpu.sync_copy(
        pout_v,
        pout_hbm.at[pl.ds(pl.multiple_of(wid * (_BPW // 8), 8), _BPW // 8)])


def _tc_body(p_ref, o_ref):
    r = lax.broadcasted_iota(jnp.int32, (128, 8), 0)
    c = lax.broadcasted_iota(jnp.int32, (128, 8), 1)
    sel = (r // 16 == c).astype(jnp.float32)
    o_ref[...] = jnp.dot(p_ref[...], sel, preferred_element_type=jnp.float32)


def kernel(inputs, user_embedding, item_embedding):
    mesh = plsc.VectorSubcoreMesh(core_axis_name="c", subcore_axis_name="s")
    conv_run = functools.partial(
        pl.kernel,
        out_type=(
            jax.ShapeDtypeStruct((_NT // 2, 2 * _D), jnp.float32),
            jax.ShapeDtypeStruct((_NT // 2, 2 * _D), jnp.float32),
        ),
        mesh=mesh,
        scratch_types=[
            pltpu.VMEM((_CROWS, _D), jnp.float32),
            pltpu.VMEM((_CROWS, _D), jnp.float32),
            pltpu.VMEM((_CROWS // 2, 2 * _D), jnp.float32),
            pltpu.VMEM((_CROWS // 2, 2 * _D), jnp.float32),
            pltpu.SemaphoreType.DMA,
            pltpu.SemaphoreType.DMA,
            pltpu.SemaphoreType.DMA,
            pltpu.SemaphoreType.DMA,
        ],
    )(_conv_body)
    sc_run = functools.partial(
        pl.kernel,
        out_type=jax.ShapeDtypeStruct((_B // 8, 128), jnp.float32),
        mesh=mesh,
        scratch_types=[
            pltpu.VMEM((_BPW * 2,), jnp.int32),
            pltpu.VMEM((_BPW,), jnp.int32),
            pltpu.VMEM((_BPW,), jnp.int32),
            pltpu.VMEM((_BPW,), jnp.int32),
            pltpu.VMEM((_BPW,), jnp.int32),
            pltpu.VMEM((_CHUNK, 2 * _D), jnp.float32),
            pltpu.VMEM((_CHUNK, 2 * _D), jnp.float32),
            pltpu.VMEM((_BPW // 8, 128), jnp.float32),
            pltpu.SemaphoreType.DMA,
        ],
    )(_sc_body)
    user2, item2 = conv_run(user_embedding, item_embedding)
    partials = sc_run(inputs.reshape(_B * 2), user2, item2)
    out = pl.pallas_call(
        _tc_body,
        out_shape=jax.ShapeDtypeStruct((_B // 8, 8), jnp.float32),
    )(partials)
    return out.reshape(_B)


# speedup vs baseline: 1.7228x; 1.4010x over previous
import functools

import jax
import jax.numpy as jnp
from jax import lax
from jax.experimental import pallas as pl
from jax.experimental.pallas import tpu as pltpu
from jax.experimental.pallas import tpu_sc as plsc

_B = 16384
_D = 64
_NW = 32
_BPW = _B // _NW
_CHUNK = 128
_NCHUNK = _BPW // _CHUNK
_NT = 100000


_CROWS = 80
_NCC = _NT // _CROWS
_CFULL = 40


def _compact(stage_v, cout_v):
    def pairs(pp, carry2):
        for q in range(4):
            p = pp * 4 + q
            for k in range(_D // 16):
                cout_v[p, pl.ds(k * 16, 16)] = \
                    stage_v[2 * p, pl.ds(k * 16, 16)]
                cout_v[p, pl.ds(_D + k * 16, 16)] = \
                    stage_v[2 * p + 1, pl.ds(k * 16, 16)]
        return carry2

    lax.fori_loop(0, _CROWS // 8, pairs, 0)


def _conv_one(table_hbm, out_hbm, nc, cb, stage, cout, sem_in, sem_out):
    def in_copy(c, b):
        g = cb + c
        return pltpu.make_async_copy(
            table_hbm.at[pl.ds(pl.multiple_of(g * _CROWS, 8), _CROWS)],
            stage[b], sem_in[b])

    def out_copy(c, b):
        g = cb + c
        return pltpu.make_async_copy(
            cout[b],
            out_hbm.at[pl.ds(pl.multiple_of(g * (_CROWS // 2), 8),
                             _CROWS // 2)],
            sem_out[b])

    in_copy(0, 0).start()

    def pair(j, carry):
        for b in range(2):
            c = 2 * j + b

            @pl.when(c < nc)
            def _():
                in_copy(c, b).wait()

                @pl.when(c + 1 < nc)
                def _():
                    in_copy(c + 1, 1 - b).start()

                @pl.when(c >= 2)
                def _():
                    out_copy(c - 2, b).wait()

                _compact(stage[b], cout[b])
                out_copy(c, b).start()
        return carry

    lax.fori_loop(0, _CFULL // 2, pair, 0)
    out_copy(0, 0).wait()
    out_copy(1, 1).wait()


def _conv_body(user_hbm, item_hbm, user2_hbm, item2_hbm,
               stage0, stage1, cout0, cout1,
               sem_in0, sem_in1, sem_out0, sem_out1):
    wid = lax.axis_index("s") * 2 + lax.axis_index("c")
    extra = _NCC - _NW * (_CFULL - 1)
    nc = jnp.where(wid < extra, _CFULL, _CFULL - 1)
    cb = wid * (_CFULL - 1) + jnp.minimum(wid, extra)
    stage = (stage0, stage1)
    cout = (cout0, cout1)
    sem_in = (sem_in0, sem_in1)
    sem_out = (sem_out0, sem_out1)
    _conv_one(user_hbm, user2_hbm, nc, cb, stage, cout, sem_in, sem_out)
    _conv_one(item_hbm, item2_hbm, nc, cb, stage, cout, sem_in, sem_out)


def _sc_body(inp_hbm, user_hbm, item_hbm, pout_hbm,
             inp_v, uidx_v, iidx_v, uoff_v, ioff_v,
             urows0_v, urows1_v, irows0_v, irows1_v,
             pout_v, sem0, sem1):
    wid = lax.axis_index("s") * 2 + lax.axis_index("c")
    base = wid * _BPW
    lane = lax.iota(jnp.int32, 16)
    lo = lane < 8
    idx_a = jnp.where(lo, lane * 2, 0)
    idx_b = jnp.where(lo, 0, (lane - 8) * 2)

    pltpu.sync_copy(
        inp_hbm.at[pl.ds(pl.multiple_of(base * 2, 8), _BPW * 2)], inp_v)

    def deint(j, carry):
        a = inp_v[pl.ds(j * 32, 16)]
        b = inp_v[pl.ds(j * 32 + 16, 16)]
        ua = a.at[idx_a].get(mode="promise_in_bounds")
        ub = b.at[idx_b].get(mode="promise_in_bounds")
        ia = a.at[idx_a + 1].get(mode="promise_in_bounds")
        ib = b.at[idx_b + 1].get(mode="promise_in_bounds")
        u = jnp.where(lo, ua, ub)
        i = jnp.where(lo, ia, ib)
        uidx_v[pl.ds(j * 16, 16)] = u >> 1
        iidx_v[pl.ds(j * 16, 16)] = i >> 1
        uoff_v[pl.ds(j * 16, 16)] = (u & 1) * _D
        ioff_v[pl.ds(j * 16, 16)] = (i & 1) * _D
        return carry

    lax.fori_loop(0, _BPW // 16, deint, 0)

    urows = (urows0_v, urows1_v)
    irows = (irows0_v, irows1_v)
    sems = (sem0, sem1)

    def copies(c, b):
        return (
            pltpu.make_async_copy(
                user_hbm.at[uidx_v.at[pl.ds(c * _CHUNK, _CHUNK)]],
                urows[b], sems[b]),
            pltpu.make_async_copy(
                item_hbm.at[iidx_v.at[pl.ds(c * _CHUNK, _CHUNK)]],
                irows[b], sems[b]),
        )

    for cp in copies(0, 0):
        cp.start()
    for c in range(_NCHUNK):
        b = c % 2
        if c + 1 < _NCHUNK:
            for cp in copies(c + 1, 1 - b):
                cp.start()
        for cp in copies(c, b):
            cp.wait()

        def body(g, carry2, c=c, b=b):
            uoffs = uoff_v[pl.ds(c * _CHUNK + g * 16, 16)]
            ioffs = ioff_v[pl.ds(c * _CHUNK + g * 16, 16)]
            for rr in range(16):
                r = g * 16 + rr
                uoff = uoffs[rr]
                ioff = ioffs[rr]
                s = (urows[b][r, pl.ds(uoff, 16)] *
                     irows[b][r, pl.ds(ioff, 16)])
                for k in range(1, _D // 16):
                    s = s + (urows[b][r, pl.ds(uoff + k * 16, 16)] *
                             irows[b][r, pl.ds(ioff + k * 16, 16)])
                row = c * 16 + g * 2 + rr // 8
                pout_v[row, pl.ds((rr % 8) * 16, 16)] = s
            return carry2

        lax.fori_loop(0, _CHUNK // 16, body, 0)
    pltpu.sync_copy(
        pout_v,
        pout_hbm.at[pl.ds(pl.multiple_of(wid * (_BPW // 8), 8), _BPW // 8)])


def _tc_body(p_ref, o_ref):
    r = lax.broadcasted_iota(jnp.int32, (128, 8), 0)
    c = lax.broadcasted_iota(jnp.int32, (128, 8), 1)
    sel = (r // 16 == c).astype(jnp.float32)
    o_ref[...] = jnp.dot(p_ref[...], sel, preferred_element_type=jnp.float32)


def kernel(inputs, user_embedding, item_embedding):
    mesh = plsc.VectorSubcoreMesh(core_axis_name="c", subcore_axis_name="s")
    conv_run = functools.partial(
        pl.kernel,
        out_type=(
            jax.ShapeDtypeStruct((_NT // 2, 2 * _D), jnp.float32),
            jax.ShapeDtypeStruct((_NT // 2, 2 * _D), jnp.float32),
        ),
        mesh=mesh,
        scratch_types=[
            pltpu.VMEM((_CROWS, _D), jnp.float32),
            pltpu.VMEM((_CROWS, _D), jnp.float32),
            pltpu.VMEM((_CROWS // 2, 2 * _D), jnp.float32),
            pltpu.VMEM((_CROWS // 2, 2 * _D), jnp.float32),
            pltpu.SemaphoreType.DMA,
            pltpu.SemaphoreType.DMA,
            pltpu.SemaphoreType.DMA,
            pltpu.SemaphoreType.DMA,
        ],
    )(_conv_body)
    sc_run = functools.partial(
        pl.kernel,
        out_type=jax.ShapeDtypeStruct((_B // 8, 128), jnp.float32),
        mesh=mesh,
        scratch_types=[
            pltpu.VMEM((_BPW * 2,), jnp.int32),
            pltpu.VMEM((_BPW,), jnp.int32),
            pltpu.VMEM((_BPW,), jnp.int32),
            pltpu.VMEM((_BPW,), jnp.int32),
            pltpu.VMEM((_BPW,), jnp.int32),
            pltpu.VMEM((_CHUNK, 2 * _D), jnp.float32),
            pltpu.VMEM((_CHUNK, 2 * _D), jnp.float32),
            pltpu.VMEM((_CHUNK, 2 * _D), jnp.float32),
            pltpu.VMEM((_CHUNK, 2 * _D), jnp.float32),
            pltpu.VMEM((_BPW // 8, 128), jnp.float32),
            pltpu.SemaphoreType.DMA,
            pltpu.SemaphoreType.DMA,
        ],
    )(_sc_body)
    user2 = user_embedding.reshape(_NT // 2, 2 * _D)
    item2 = item_embedding.reshape(_NT // 2, 2 * _D)
    partials = sc_run(inputs.reshape(_B * 2), user2, item2)
    out = pl.pallas_call(
        _tc_body,
        out_shape=jax.ShapeDtypeStruct((_B // 8, 8), jnp.float32),
    )(partials)
    return out.reshape(_B)


# speedup vs baseline: 1.8227x; 1.0579x over previous
import functools

import jax
import jax.numpy as jnp
from jax import lax
from jax.experimental import pallas as pl
from jax.experimental.pallas import tpu as pltpu
from jax.experimental.pallas import tpu_sc as plsc

_B = 16384
_D = 64
_NW = 32
_BPW = _B // _NW
_CHUNK = 128
_NCHUNK = _BPW // _CHUNK


def _sc_body(uidx_hbm, iidx_hbm, user_hbm, item_hbm, pout_hbm,
             uidx_v, iidx_v, urows_v, irows_v, pout_v, sem):
    wid = lax.axis_index("s") * 2 + lax.axis_index("c")
    base = wid * _BPW

    pltpu.sync_copy(uidx_hbm.at[pl.ds(base, _BPW)], uidx_v)
    pltpu.sync_copy(iidx_hbm.at[pl.ds(base, _BPW)], iidx_v)

    copies = []
    for j in range(_NCHUNK):
        idx = uidx_v.at[pl.ds(j * _CHUNK, _CHUNK)]
        dst = urows_v.at[pl.ds(j * _CHUNK, _CHUNK)]
        copies.append(pltpu.make_async_copy(user_hbm.at[idx], dst, sem))
    for j in range(_NCHUNK):
        idx = iidx_v.at[pl.ds(j * _CHUNK, _CHUNK)]
        dst = irows_v.at[pl.ds(j * _CHUNK, _CHUNK)]
        copies.append(pltpu.make_async_copy(item_hbm.at[idx], dst, sem))
    for c in copies:
        c.start()
    for c in copies:
        c.wait()

    def body(g, carry):
        for rr in range(4):
            r = g * 4 + rr
            s = urows_v[r, pl.ds(0, 16)] * irows_v[r, pl.ds(0, 16)]
            for k in range(1, _D // 16):
                s = s + urows_v[r, pl.ds(k * 16, 16)] * irows_v[r, pl.ds(k * 16, 16)]
            pout_v[r, pl.ds(0, 16)] = s
        return carry

    lax.fori_loop(0, _BPW // 4, body, 0)
    pltpu.sync_copy(pout_v, pout_hbm.at[pl.ds(base, _BPW)])


def _tc_body(p_ref, o_ref):
    r = lax.broadcasted_iota(jnp.int32, (128, 8), 0)
    c = lax.broadcasted_iota(jnp.int32, (128, 8), 1)
    sel = (r // 16 == c).astype(jnp.float32)
    o_ref[...] = jnp.dot(p_ref[...], sel, preferred_element_type=jnp.float32)


def kernel(inputs, user_embedding, item_embedding):
    mesh = plsc.VectorSubcoreMesh(core_axis_name="c", subcore_axis_name="s")
    sc_run = functools.partial(
        pl.kernel,
        out_type=jax.ShapeDtypeStruct((_B, 16), jnp.float32),
        mesh=mesh,
        compiler_params=pltpu.CompilerParams(use_tc_tiling_on_sc=False),
        scratch_types=[
            pltpu.VMEM((_BPW,), jnp.int32),
            pltpu.VMEM((_BPW,), jnp.int32),
            pltpu.VMEM((_BPW, _D), jnp.float32),
            pltpu.VMEM((_BPW, _D), jnp.float32),
            pltpu.VMEM((_BPW, 16), jnp.float32),
            pltpu.SemaphoreType.DMA,
        ],
    )(_sc_body)
    uidx = inputs[:, 0].reshape(_B)
    iidx = inputs[:, 1].reshape(_B)
    partials = sc_run(uidx, iidx, user_embedding, item_embedding)
    out = pl.pallas_call(
        _tc_body,
        out_shape=jax.ShapeDtypeStruct((_B // 8, 8), jnp.float32),
    )(partials.reshape(_B // 8, 128))
    return out.reshape(_B)
